# baseline (speedup 1.0000x reference)
import functools

import jax
import jax.numpy as jnp
from jax import lax
from jax.experimental import pallas as pl
from jax.experimental.pallas import tpu as pltpu
from jax.experimental.pallas import tpu_sc as plsc

NC = 2
NS = 16
L = 16
NW = NC * NS
B = 16384
F = 64
W = 2 * F
BPW = B // NW
CH = 128
NCH = BPW // CH
NU = 100000
NM = 100000


def _mf_body(user_r, movie_r, ut_r, mt_r, but_r, bmt_r, bias_r, out_r,
             uidx, midx, gu, gm, pu, pm, urows, mrows, bu, bm, ovec, bvec,
             sbuf, sem, bsem):
    wid = lax.axis_index("s") * NC + lax.axis_index("c")
    base = wid * BPW

    pltpu.sync_copy(user_r.at[pl.ds(wid * NCH, NCH)], uidx)
    pltpu.sync_copy(movie_r.at[pl.ds(wid * NCH, NCH)], midx)
    pltpu.sync_copy(bias_r, bvec)

    for j in range(NCH):
        for k in range(CH // L):
            uv = uidx[j, pl.ds(k * L, L)]
            gu[j, pl.ds(k * L, L)] = lax.shift_right_logical(uv, 1)
            pu[pl.ds(j * CH + k * L, L)] = lax.bitwise_and(uv, 1)
            mv = midx[j, pl.ds(k * L, L)]
            gm[j, pl.ds(k * L, L)] = lax.shift_right_logical(mv, 1)
            pm[pl.ds(j * CH + k * L, L)] = lax.bitwise_and(mv, 1)

    bcopies = []
    for j in range(NCH):
        bcopies.append(pltpu.async_copy(
            but_r.at[uidx.at[j]], bu.at[pl.ds(j * CH, CH)], bsem))
        bcopies.append(pltpu.async_copy(
            bmt_r.at[midx.at[j]], bm.at[pl.ds(j * CH, CH)], bsem))

    def fire(j, buf):
        cu = pltpu.async_copy(ut_r.at[gu.at[j]], urows.at[buf], sem)
        cm = pltpu.async_copy(mt_r.at[gm.at[j]], mrows.at[buf], sem)
        return cu, cm

    pend = fire(0, 0)

    for j in range(NCH):
        pend[0].wait()
        pend[1].wait()
        cur = j % 2
        if j + 1 < NCH:
            pend = fire(j + 1, (j + 1) % 2)
        lanes = lax.iota(jnp.int32, L)
        for g in range(CH // L):
            pu16 = pu[pl.ds(j * CH + g * L, L)]
            pm16 = pm[pl.ds(j * CH + g * L, L)]
            for r in range(L):
                i = g * L + r
                uoff = pu16[r] * F
                moff = pm16[r] * F
                s = (urows[cur, i, pl.ds(uoff, L)]
                     * mrows[cur, i, pl.ds(moff, L)])
                s = s + (urows[cur, i, pl.ds(uoff + L, L)]
                         * mrows[cur, i, pl.ds(moff + L, L)])
                s = s + (urows[cur, i, pl.ds(uoff + 2 * L, L)]
                         * mrows[cur, i, pl.ds(moff + 2 * L, L)])
                s = s + (urows[cur, i, pl.ds(uoff + 3 * L, L)]
                         * mrows[cur, i, pl.ds(moff + 3 * L, L)])
                plsc.store_scatter(
                    sbuf, [lanes, jnp.full((L,), r, jnp.int32)], s)
            acc = sbuf[0, pl.ds(0, L)]
            for l in range(1, L):
                acc = acc + sbuf[l, pl.ds(0, L)]
            ovec[pl.ds(j * CH + g * L, L)] = acc

    for c in bcopies:
        c.wait()

    bias_v = bvec[...]
    for g in range(BPW // L):
        sl = pl.ds(g * L, L)
        ovec[sl] = ovec[sl] + bu[sl] + bm[sl] + bias_v

    pltpu.sync_copy(ovec, out_r.at[pl.ds(base, BPW)])


@functools.partial(
    pl.kernel,
    out_type=jax.ShapeDtypeStruct((B,), jnp.float32),
    mesh=plsc.VectorSubcoreMesh(
        core_axis_name="c", subcore_axis_name="s",
        num_cores=NC, num_subcores=NS),
    compiler_params=pltpu.CompilerParams(
        needs_layout_passes=False, use_tc_tiling_on_sc=True),
    scratch_types=[
        pltpu.VMEM((NCH, CH), jnp.int32),
        pltpu.VMEM((NCH, CH), jnp.int32),
        pltpu.VMEM((NCH, CH), jnp.int32),
        pltpu.VMEM((NCH, CH), jnp.int32),
        pltpu.VMEM((BPW,), jnp.int32),
        pltpu.VMEM((BPW,), jnp.int32),
        pltpu.VMEM((2, CH, W), jnp.float32),
        pltpu.VMEM((2, CH, W), jnp.float32),
        pltpu.VMEM((BPW,), jnp.float32),
        pltpu.VMEM((BPW,), jnp.float32),
        pltpu.VMEM((BPW,), jnp.float32),
        pltpu.VMEM((L,), jnp.float32),
        pltpu.VMEM((L, L), jnp.float32),
        pltpu.SemaphoreType.DMA,
        pltpu.SemaphoreType.DMA,
    ],
)
def _mf_kernel(user_r, movie_r, ut_r, mt_r, but_r, bmt_r, bias_r, out_r,
               uidx, midx, gu, gm, pu, pm, urows, mrows, bu, bm, ovec, bvec,
               sbuf, sem, bsem):
    _mf_body(user_r, movie_r, ut_r, mt_r, but_r, bmt_r, bias_r, out_r,
             uidx, midx, gu, gm, pu, pm, urows, mrows, bu, bm, ovec, bvec,
             sbuf, sem, bsem)


@jax.jit
def kernel(user, movie, user_table, movie_table, bias_user_table,
           bias_movie_table, bias):
    user2 = user.astype(jnp.int32).reshape(B // CH, CH)
    movie2 = movie.astype(jnp.int32).reshape(B // CH, CH)
    bias16 = jnp.broadcast_to(bias.astype(jnp.float32), (L,))
    ut128 = user_table[:NU].reshape(NU // 2, W)
    mt128 = movie_table.reshape(NM // 2, W)
    return _mf_kernel(user2, movie2, ut128, mt128,
                      bias_user_table.reshape(-1),
                      bias_movie_table.reshape(-1), bias16)


# speedup vs baseline: 1.2352x; 1.2352x over previous
import functools

import jax
import jax.numpy as jnp
from jax import lax
from jax.experimental import pallas as pl
from jax.experimental.pallas import tpu as pltpu
from jax.experimental.pallas import tpu_sc as plsc

NC = 2
NS = 16
L = 16
B = 16384
F = 64
NU = 100001
NM = 100000
BS = B // NS
CH = 128
NCHT = BS // CH
FIT = 4
NIT = (F // NC) // FIT


def _mf_body(user_r, movie_r, utt_r, mtt_r, but_r, bmt_r, bias_r, out_r,
             uidx, midx, uval, mval, psum, bu, bm, bvec,
             ubufs, mbufs, gsem, bsem):
    c = lax.axis_index("c")
    s = lax.axis_index("s")
    base = s * BS

    pltpu.sync_copy(user_r.at[pl.ds(s * NCHT, NCHT)], uidx)
    pltpu.sync_copy(movie_r.at[pl.ds(s * NCHT, NCHT)], midx)
    pltpu.sync_copy(bias_r, bvec)

    bcopies = []

    @pl.when(c == 0)
    def _():
        for j in range(NCHT):
            bcopies.append(pltpu.async_copy(
                but_r.at[uidx.at[j]], bu.at[pl.ds(j * CH, CH)], bsem))
            bcopies.append(pltpu.async_copy(
                bmt_r.at[midx.at[j]], bm.at[pl.ds(j * CH, CH)], bsem))

    for it in range(NIT):
        @pl.when(s == 0)
        def _(it=it):
            for half, fbase in ((0, 0), (1, F // NC)):
                @pl.when(c == half)
                def _(it=it, fbase=fbase):
                    for k in range(FIT):
                        f = fbase + it * FIT + k
                        pltpu.sync_copy(utt_r.at[f], ubufs[k])
                        pltpu.sync_copy(mtt_r.at[f], mbufs[k])

        plsc.subcore_barrier()

        for k in range(FIT):
            def fire(j, carry, k=k):
                pltpu.async_copy(
                    ubufs[k].at[uidx.at[j]],
                    uval.at[k, pl.ds(j * CH, CH)], gsem)
                pltpu.async_copy(
                    mbufs[k].at[midx.at[j]],
                    mval.at[k, pl.ds(j * CH, CH)], gsem)
                return carry
            lax.fori_loop(0, NCHT, fire, 0)

        def drain(j, carry):
            pltpu.make_async_copy(
                utt_r.at[0, pl.ds(0, CH)], uval.at[0, pl.ds(0, CH)],
                gsem).wait()
            return carry
        lax.fori_loop(0, 2 * FIT * NCHT, drain, 0)

        def fma(sl, carry, it=it):
            ds = pl.ds(sl * L, L)
            acc = uval[0, ds] * mval[0, ds]
            for k in range(1, FIT):
                acc = acc + uval[k, ds] * mval[k, ds]
            if it > 0:
                acc = acc + psum[ds]
            psum[ds] = acc
            return carry
        lax.fori_loop(0, BS // L, fma, 0)

        plsc.subcore_barrier()

    @pl.when(c == 0)
    def _():
        for g in bcopies:
            g.wait()
        bias_v = bvec[...]

        def badd(sl, carry):
            ds = pl.ds(sl * L, L)
            psum[ds] = psum[ds] + bu[ds] + bm[ds] + bias_v
            return carry
        lax.fori_loop(0, BS // L, badd, 0)

    pltpu.sync_copy(psum, out_r.at[c, pl.ds(base, BS)])


@functools.partial(
    pl.kernel,
    out_type=jax.ShapeDtypeStruct((NC, B), jnp.float32),
    mesh=plsc.VectorSubcoreMesh(
        core_axis_name="c", subcore_axis_name="s",
        num_cores=NC, num_subcores=NS),
    compiler_params=pltpu.CompilerParams(
        needs_layout_passes=False, use_tc_tiling_on_sc=True),
    scratch_types=[
        pltpu.VMEM((NCHT, CH), jnp.int32),
        pltpu.VMEM((NCHT, CH), jnp.int32),
        pltpu.VMEM((FIT, BS), jnp.float32),
        pltpu.VMEM((FIT, BS), jnp.float32),
        pltpu.VMEM((BS,), jnp.float32),
        pltpu.VMEM((BS,), jnp.float32),
        pltpu.VMEM((BS,), jnp.float32),
        pltpu.VMEM((L,), jnp.float32),
        [pltpu.VMEM_SHARED((NU,), jnp.float32) for _ in range(FIT)],
        [pltpu.VMEM_SHARED((NM,), jnp.float32) for _ in range(FIT)],
        pltpu.SemaphoreType.DMA,
        pltpu.SemaphoreType.DMA,
    ],
)
def _mf_kernel(user_r, movie_r, utt_r, mtt_r, but_r, bmt_r, bias_r, out_r,
               uidx, midx, uval, mval, psum, bu, bm, bvec,
               ubufs, mbufs, gsem, bsem):
    _mf_body(user_r, movie_r, utt_r, mtt_r, but_r, bmt_r, bias_r, out_r,
             uidx, midx, uval, mval, psum, bu, bm, bvec,
             ubufs, mbufs, gsem, bsem)


@jax.jit
def kernel(user, movie, user_table, movie_table, bias_user_table,
           bias_movie_table, bias):
    user2 = user.astype(jnp.int32).reshape(B // CH, CH)
    movie2 = movie.astype(jnp.int32).reshape(B // CH, CH)
    bias16 = jnp.broadcast_to(bias.astype(jnp.float32), (L,))
    parts = _mf_kernel(user2, movie2, user_table.T, movie_table.T,
                       bias_user_table.reshape(-1),
                       bias_movie_table.reshape(-1), bias16)
    return parts[0] + parts[1]


# speedup vs baseline: 1.7216x; 1.3938x over previous
import functools

import jax
import jax.numpy as jnp
from jax import lax
from jax.experimental import pallas as pl
from jax.experimental.pallas import tpu as pltpu
from jax.experimental.pallas import tpu_sc as plsc

NC = 2
NS = 16
L = 16
B = 16384
F = 64
NU = 100001
NM = 100000
BS = B // NS
CH = 128
NCHT = BS // CH
FIT = 4
NIT = (F // NC) // FIT


def _mf_body(user_r, movie_r, utt_r, mtt_r, but_r, bmt_r, bias_r, out_r,
             uidx, midx, uval, mval, psum, bu, bm, bvec,
             ubufs, mbufs, ssem, gsem, bsem):
    c = lax.axis_index("c")
    s = lax.axis_index("s")
    base = s * BS

    pltpu.sync_copy(user_r.at[pl.ds(s * NCHT, NCHT)], uidx)
    pltpu.sync_copy(movie_r.at[pl.ds(s * NCHT, NCHT)], midx)
    pltpu.sync_copy(bias_r, bvec)

    bcopies = []

    @pl.when(c == 0)
    def _():
        for j in range(NCHT):
            bcopies.append(pltpu.async_copy(
                but_r.at[uidx.at[j]], bu.at[pl.ds(j * CH, CH)], bsem))
            bcopies.append(pltpu.async_copy(
                bmt_r.at[midx.at[j]], bm.at[pl.ds(j * CH, CH)], bsem))

    for it in range(NIT):
        @pl.when(s == 0)
        def _(it=it):
            for half, fbase in ((0, 0), (1, F // NC)):
                @pl.when(c == half)
                def _(it=it, fbase=fbase):
                    scopies = []
                    for k in range(FIT):
                        f = fbase + it * FIT + k
                        scopies.append(
                            pltpu.async_copy(utt_r.at[f], ubufs[k], ssem))
                        scopies.append(
                            pltpu.async_copy(mtt_r.at[f], mbufs[k], ssem))
                    for sc in scopies:
                        sc.wait()

        plsc.subcore_barrier()

        for k in range(FIT):
            def fire(j, carry, k=k):
                pltpu.async_copy(
                    ubufs[k].at[uidx.at[j]],
                    uval.at[k, pl.ds(j * CH, CH)], gsem)
                pltpu.async_copy(
                    mbufs[k].at[midx.at[j]],
                    mval.at[k, pl.ds(j * CH, CH)], gsem)
                return carry
            lax.fori_loop(0, NCHT, fire, 0)

        def drain(j, carry):
            pltpu.make_async_copy(
                utt_r.at[0, pl.ds(0, CH)], uval.at[0, pl.ds(0, CH)],
                gsem).wait()
            return carry
        lax.fori_loop(0, 2 * FIT * NCHT, drain, 0)

        def fma(sl, carry, it=it):
            ds = pl.ds(sl * L, L)
            acc = uval[0, ds] * mval[0, ds]
            for k in range(1, FIT):
                acc = acc + uval[k, ds] * mval[k, ds]
            if it > 0:
                acc = acc + psum[ds]
            psum[ds] = acc
            return carry
        lax.fori_loop(0, BS // L, fma, 0)

        plsc.subcore_barrier()

    @pl.when(c == 0)
    def _():
        for g in bcopies:
            g.wait()
        bias_v = bvec[...]

        def badd(sl, carry):
            ds = pl.ds(sl * L, L)
            psum[ds] = psum[ds] + bu[ds] + bm[ds] + bias_v
            return carry
        lax.fori_loop(0, BS // L, badd, 0)

    pltpu.sync_copy(psum, out_r.at[c, pl.ds(base, BS)])


@functools.partial(
    pl.kernel,
    out_type=jax.ShapeDtypeStruct((NC, B), jnp.float32),
    mesh=plsc.VectorSubcoreMesh(
        core_axis_name="c", subcore_axis_name="s",
        num_cores=NC, num_subcores=NS),
    compiler_params=pltpu.CompilerParams(
        needs_layout_passes=False, use_tc_tiling_on_sc=True),
    scratch_types=[
        pltpu.VMEM((NCHT, CH), jnp.int32),
        pltpu.VMEM((NCHT, CH), jnp.int32),
        pltpu.VMEM((FIT, BS), jnp.float32),
        pltpu.VMEM((FIT, BS), jnp.float32),
        pltpu.VMEM((BS,), jnp.float32),
        pltpu.VMEM((BS,), jnp.float32),
        pltpu.VMEM((BS,), jnp.float32),
        pltpu.VMEM((L,), jnp.float32),
        [pltpu.VMEM_SHARED((NU,), jnp.float32) for _ in range(FIT)],
        [pltpu.VMEM_SHARED((NM,), jnp.float32) for _ in range(FIT)],
        pltpu.SemaphoreType.DMA,
        pltpu.SemaphoreType.DMA,
        pltpu.SemaphoreType.DMA,
    ],
)
def _mf_kernel(user_r, movie_r, utt_r, mtt_r, but_r, bmt_r, bias_r, out_r,
               uidx, midx, uval, mval, psum, bu, bm, bvec,
               ubufs, mbufs, ssem, gsem, bsem):
    _mf_body(user_r, movie_r, utt_r, mtt_r, but_r, bmt_r, bias_r, out_r,
             uidx, midx, uval, mval, psum, bu, bm, bvec,
             ubufs, mbufs, ssem, gsem, bsem)


@jax.jit
def kernel(user, movie, user_table, movie_table, bias_user_table,
           bias_movie_table, bias):
    user2 = user.astype(jnp.int32).reshape(B // CH, CH)
    movie2 = movie.astype(jnp.int32).reshape(B // CH, CH)
    bias16 = jnp.broadcast_to(bias.astype(jnp.float32), (L,))
    parts = _mf_kernel(user2, movie2, user_table.T, movie_table.T,
                       bias_user_table.reshape(-1),
                       bias_movie_table.reshape(-1), bias16)
    return parts[0] + parts[1]


# speedup vs baseline: 2.0382x; 1.1839x over previous
import functools

import jax
import jax.numpy as jnp
from jax import lax
from jax.experimental import pallas as pl
from jax.experimental.pallas import tpu as pltpu
from jax.experimental.pallas import tpu_sc as plsc

NC = 2
NS = 16
L = 16
B = 16384
F = 64
NU = 100001
NM = 100000
BS = B // NS
CH = 128
NCHT = BS // CH
FIT = 4
NIT = (F // NC) // FIT


def _mf_body(user_r, movie_r, utt_r, mtt_r, but_r, bmt_r, bias_r, out_r,
             uidx, midx, uval, mval, psum, bvec,
             ubufs, mbufs, ssem, gsem, bsem):
    c = lax.axis_index("c")
    s = lax.axis_index("s")
    base = s * BS

    pltpu.sync_copy(user_r.at[pl.ds(s * NCHT, NCHT)], uidx)
    pltpu.sync_copy(movie_r.at[pl.ds(s * NCHT, NCHT)], midx)
    pltpu.sync_copy(bias_r, bvec)


    stage_descs = {}

    def fire_stage(it):
        buf = it % 2

        @pl.when(s == 0)
        def _():
            for half, fbase in ((0, 0), (1, F // NC)):
                @pl.when(c == half)
                def _(fbase=fbase):
                    descs = []
                    for k in range(FIT):
                        f = fbase + it * FIT + k
                        descs.append(pltpu.async_copy(
                            utt_r.at[f], ubufs[buf][k], ssem))
                        descs.append(pltpu.async_copy(
                            mtt_r.at[f], mbufs[buf][k], ssem))
                    stage_descs[(it, half)] = descs

    def wait_stage(it):
        @pl.when(s == 0)
        def _():
            for half in (0, 1):
                @pl.when(c == half)
                def _(half=half):
                    for d in stage_descs[(it, half)]:
                        d.wait()

    fire_stage(0)

    for it in range(NIT):
        buf = it % 2
        wait_stage(it)
        plsc.subcore_barrier()
        if it + 1 < NIT:
            fire_stage(it + 1)

        for k in range(FIT):
            def fire(j, carry, k=k, buf=buf):
                pltpu.async_copy(
                    ubufs[buf][k].at[uidx.at[j]],
                    uval.at[k, pl.ds(j * CH, CH)], gsem)
                pltpu.async_copy(
                    mbufs[buf][k].at[midx.at[j]],
                    mval.at[k, pl.ds(j * CH, CH)], gsem)
                return carry
            lax.fori_loop(0, NCHT, fire, 0)

        def drain(j, carry):
            pltpu.make_async_copy(
                utt_r.at[0, pl.ds(0, CH)], uval.at[0, pl.ds(0, CH)],
                gsem).wait()
            return carry
        lax.fori_loop(0, 2 * FIT * NCHT, drain, 0)

        def fma(sl, carry, it=it):
            ds = pl.ds(sl * L, L)
            acc = uval[0, ds] * mval[0, ds]
            for k in range(1, FIT):
                acc = acc + uval[k, ds] * mval[k, ds]
            if it > 0:
                acc = acc + psum[ds]
            psum[ds] = acc
            return carry
        lax.fori_loop(0, BS // L, fma, 0)

        plsc.subcore_barrier()

    @pl.when(c == 0)
    def _():
        bcopies = []
        for j in range(NCHT):
            bcopies.append(pltpu.async_copy(
                but_r.at[uidx.at[j]], uval.at[0, pl.ds(j * CH, CH)], bsem))
            bcopies.append(pltpu.async_copy(
                bmt_r.at[midx.at[j]], mval.at[0, pl.ds(j * CH, CH)], bsem))
        for g in bcopies:
            g.wait()
        bias_v = bvec[...]

        def badd(sl, carry):
            ds = pl.ds(sl * L, L)
            psum[ds] = psum[ds] + uval[0, ds] + mval[0, ds] + bias_v
            return carry
        lax.fori_loop(0, BS // L, badd, 0)

    pltpu.sync_copy(psum, out_r.at[c, pl.ds(base, BS)])


@functools.partial(
    pl.kernel,
    out_type=jax.ShapeDtypeStruct((NC, B), jnp.float32),
    mesh=plsc.VectorSubcoreMesh(
        core_axis_name="c", subcore_axis_name="s",
        num_cores=NC, num_subcores=NS),
    compiler_params=pltpu.CompilerParams(
        needs_layout_passes=False, use_tc_tiling_on_sc=True),
    scratch_types=[
        pltpu.VMEM((NCHT, CH), jnp.int32),
        pltpu.VMEM((NCHT, CH), jnp.int32),
        pltpu.VMEM((FIT, BS), jnp.float32),
        pltpu.VMEM((FIT, BS), jnp.float32),
        pltpu.VMEM((BS,), jnp.float32),
        pltpu.VMEM((L,), jnp.float32),
        [[pltpu.VMEM_SHARED((NU,), jnp.float32) for _ in range(FIT)]
         for _ in range(2)],
        [[pltpu.VMEM_SHARED((NM,), jnp.float32) for _ in range(FIT)]
         for _ in range(2)],
        pltpu.SemaphoreType.DMA,
        pltpu.SemaphoreType.DMA,
        pltpu.SemaphoreType.DMA,
    ],
)
def _mf_kernel(user_r, movie_r, utt_r, mtt_r, but_r, bmt_r, bias_r, out_r,
               uidx, midx, uval, mval, psum, bvec,
               ubufs, mbufs, ssem, gsem, bsem):
    _mf_body(user_r, movie_r, utt_r, mtt_r, but_r, bmt_r, bias_r, out_r,
             uidx, midx, uval, mval, psum, bvec,
             ubufs, mbufs, ssem, gsem, bsem)


@jax.jit
def kernel(user, movie, user_table, movie_table, bias_user_table,
           bias_movie_table, bias):
    user2 = user.astype(jnp.int32).reshape(B // CH, CH)
    movie2 = movie.astype(jnp.int32).reshape(B // CH, CH)
    bias16 = jnp.broadcast_to(bias.astype(jnp.float32), (L,))
    parts = _mf_kernel(user2, movie2, user_table.T, movie_table.T,
                       bias_user_table.reshape(-1),
                       bias_movie_table.reshape(-1), bias16)
    return parts[0] + parts[1]
